# Initial kernel scaffold; baseline (speedup 1.0000x reference)
import jax
import jax.numpy as jnp
from jax.experimental import pallas as pl


def kernel(x, edge_index, edge_weight, Wz, bz, Wr, br, Wh, bh, Wg, bg, gamma, beta, Wl, bl):
    raise NotImplementedError("write your pallas kernel here")


# speedup vs baseline: 11.0910x; 11.0910x over previous
import functools

import jax
import jax.numpy as jnp
from jax import lax
from jax.experimental import pallas as pl
from jax.experimental.pallas import tpu as pltpu
from jax.experimental.pallas import tpu_sc as plsc

N = 10000
E = 320000
F = 128
NPAD = 10240
NC = 2
NS = 16
NW = NC * NS
CH = 80
EPW = E // NW
EPT = E // NS
RPT = NPAD // NS
RPADT = NPAD // NS
NBLK = 10
BR = N // NBLK
_f32 = jnp.float32
_i32 = jnp.int32

_mesh = plsc.VectorSubcoreMesh(core_axis_name="c", subcore_axis_name="s")
_sc_params = pltpu.CompilerParams(needs_layout_passes=False)


@functools.partial(
    pl.kernel,
    mesh=_mesh,
    out_type=[jax.ShapeDtypeStruct((NC, NPAD), _f32)] * 3,
    scratch_types=[
        pltpu.VMEM((CH,), _i32),
        pltpu.VMEM((CH,), _i32),
        pltpu.VMEM((CH,), _f32),
        pltpu.VMEM((CH,), _f32),
        pltpu.VMEM_SHARED((NPAD,), _f32),
        pltpu.VMEM_SHARED((NPAD,), _f32),
        pltpu.VMEM_SHARED((NPAD,), _f32),
    ],
)
def _deg_kernel(src_h, dst_h, ew_h, zn_h, dego_h, degi_h, cnt_h,
                idx_s, idx_d, valb, oneb, acc_o, acc_i, acc_c):
    c = lax.axis_index("c")
    s = lax.axis_index("s")
    z0 = s * RPADT
    pltpu.sync_copy(zn_h.at[pl.ds(z0, RPADT)], acc_o.at[pl.ds(z0, RPADT)])
    pltpu.sync_copy(zn_h.at[pl.ds(z0, RPADT)], acc_i.at[pl.ds(z0, RPADT)])
    pltpu.sync_copy(zn_h.at[pl.ds(z0, RPADT)], acc_c.at[pl.ds(z0, RPADT)])
    for j in range(CH // 16):
        oneb[pl.ds(j * 16, 16)] = jnp.ones((16,), _f32)
    plsc.subcore_barrier()
    base = (s * NC + c) * EPW

    def chunk(i, carry):
        off = base + i * CH
        pltpu.sync_copy(src_h.at[pl.ds(off, CH)], idx_s)
        pltpu.sync_copy(dst_h.at[pl.ds(off, CH)], idx_d)
        pltpu.sync_copy(ew_h.at[pl.ds(off, CH)], valb)
        pltpu.sync_copy(valb, acc_o.at[idx_s], add=True)
        pltpu.sync_copy(valb, acc_i.at[idx_d], add=True)
        pltpu.sync_copy(oneb, acc_c.at[idx_d], add=True)
        return carry

    lax.fori_loop(0, EPW // CH, chunk, 0)
    plsc.subcore_barrier()

    @pl.when(s == 0)
    def _():
        pltpu.sync_copy(acc_o, dego_h.at[c])
        pltpu.sync_copy(acc_i, degi_h.at[c])
        pltpu.sync_copy(acc_c, cnt_h.at[c])


def _inv_body(dego_ref, degi_ref, cnt_ref, doi_ref, dii_ref, dgc_ref):
    po = dego_ref[0:1, :] + dego_ref[1:2, :]
    pi = degi_ref[0:1, :] + degi_ref[1:2, :]
    pc = cnt_ref[0:1, :] + cnt_ref[1:2, :]
    doi_ref[...] = jnp.where(po > 0, 1.0 / po, 0.0)
    dii_ref[...] = jnp.where(pi > 0, 1.0 / pi, 0.0)
    dgc_ref[...] = lax.rsqrt(pc + 1.0)


_inv_call = pl.pallas_call(
    _inv_body,
    out_shape=[jax.ShapeDtypeStruct((1, NPAD), _f32)] * 3,
)


@functools.partial(
    pl.kernel,
    mesh=_mesh,
    out_type=[jax.ShapeDtypeStruct((NPAD, F), _f32)] * 2,
    compiler_params=_sc_params,
    scratch_types=[
        pltpu.VMEM((NPAD,), _f32),
        pltpu.VMEM((CH,), _i32),
        pltpu.VMEM((CH,), _i32),
        pltpu.VMEM((CH,), _f32),
        pltpu.VMEM((CH,), _f32),
        pltpu.VMEM((CH, F), _f32),
        pltpu.VMEM_SHARED((NPAD, F), _f32),
        pltpu.SemaphoreType.DMA,
    ],
)
def _prop_kernel(x_h, src_h, dst_h, ew_h, doi_h, dii_h, zr_h, po_h, pi_h,
                 dv, gidx, sidx, wb, nb, rows, acc, sem):
    c = lax.axis_index("c")
    s = lax.axis_index("s")
    r0 = s * RPT
    pltpu.sync_copy(zr_h.at[pl.ds(r0, RPT)], acc.at[pl.ds(r0, RPT)])

    @pl.when(c == 0)
    def _():
        pltpu.sync_copy(doi_h, dv)

    @pl.when(c == 1)
    def _():
        pltpu.sync_copy(dii_h, dv)

    plsc.subcore_barrier()
    base = s * EPT

    def chunk(i, carry):
        off = base + i * CH

        @pl.when(c == 0)
        def _():
            pltpu.sync_copy(src_h.at[pl.ds(off, CH)], gidx)
            pltpu.sync_copy(dst_h.at[pl.ds(off, CH)], sidx)

        @pl.when(c == 1)
        def _():
            pltpu.sync_copy(dst_h.at[pl.ds(off, CH)], gidx)
            pltpu.sync_copy(src_h.at[pl.ds(off, CH)], sidx)

        pltpu.sync_copy(ew_h.at[pl.ds(off, CH)], wb)
        pltpu.async_copy(x_h.at[gidx], rows, sem).wait()
        for g in range(CH // 16):
            i16 = gidx[pl.ds(g * 16, 16)]
            d16 = plsc.load_gather(dv, [i16])
            nb[pl.ds(g * 16, 16)] = wb[pl.ds(g * 16, 16)] * d16

        def erow(e, carry2):
            nv = plsc.load_gather(nb, [jnp.full((16,), e, _i32)])
            for sg in range(F // 16):
                rows[e, pl.ds(sg * 16, 16)] = rows[e, pl.ds(sg * 16, 16)] * nv
            return carry2

        lax.fori_loop(0, CH, erow, 0)
        pltpu.sync_copy(rows, acc.at[sidx], add=True)
        return carry

    lax.fori_loop(0, EPT // CH, chunk, 0)
    plsc.subcore_barrier()

    @pl.when(c == 0)
    def _():
        pltpu.sync_copy(acc.at[pl.ds(r0, RPT)], po_h.at[pl.ds(r0, RPT)])

    @pl.when(c == 1)
    def _():
        pltpu.sync_copy(acc.at[pl.ds(r0, RPT)], pi_h.at[pl.ds(r0, RPT)])


def _gru_body(x_ref, po_ref, pi_ref, wz00_ref, wz10_ref, wz01_ref, wz11_ref,
              wh00_ref, wh10_ref, wh01_ref, wh11_ref, bz_ref, bh_ref,
              wg_ref, dinv_ref, out_ref):
    xb = x_ref[...]
    pob = po_ref[...]
    pib = pi_ref[...]
    az = wz00_ref[...] + wz10_ref[...]
    ah = wh00_ref[...] + wh10_ref[...]
    zpre = (jnp.dot(xb, az, preferred_element_type=_f32)
            + jnp.dot(pob, wz01_ref[...], preferred_element_type=_f32)
            + jnp.dot(pib, wz11_ref[...], preferred_element_type=_f32)
            + bz_ref[...])
    hpre = (jnp.dot(xb, ah, preferred_element_type=_f32)
            + jnp.dot(pob, wh01_ref[...], preferred_element_type=_f32)
            + jnp.dot(pib, wh11_ref[...], preferred_element_type=_f32)
            + bh_ref[...])
    z = jax.nn.sigmoid(zpre)
    ht = jnp.tanh(hpre)
    h = (1.0 - z) * ht
    out_ref[...] = dinv_ref[...] * jnp.dot(h, wg_ref[...],
                                           preferred_element_type=_f32)


_w_spec = pl.BlockSpec((F, F), lambda i: (0, 0))
_b_spec = pl.BlockSpec((1, F), lambda i: (0, 0))
_row_spec = pl.BlockSpec((BR, F), lambda i: (i, 0))
_col_spec = pl.BlockSpec((BR, 1), lambda i: (i, 0))

_gru_call = pl.pallas_call(
    _gru_body,
    grid=(NBLK,),
    in_specs=[_row_spec, _row_spec, _row_spec,
              _w_spec, _w_spec, _w_spec, _w_spec,
              _w_spec, _w_spec, _w_spec, _w_spec,
              _b_spec, _b_spec, _w_spec, _col_spec],
    out_specs=_row_spec,
    out_shape=jax.ShapeDtypeStruct((N, F), _f32),
)


@functools.partial(
    pl.kernel,
    mesh=_mesh,
    out_type=jax.ShapeDtypeStruct((NC, NPAD, F), _f32),
    scratch_types=[
        pltpu.VMEM((CH,), _i32),
        pltpu.VMEM((CH,), _i32),
        pltpu.VMEM((CH, F), _f32),
        pltpu.VMEM_SHARED((NPAD, F), _f32),
        pltpu.SemaphoreType.DMA,
    ],
)
def _gcn_kernel(hwp_h, src_h, dst_h, zr_h, agg_h, gidx, sidx, rows, acc, sem):
    c = lax.axis_index("c")
    s = lax.axis_index("s")
    r0 = s * RPT
    pltpu.sync_copy(zr_h.at[pl.ds(r0, RPT)], acc.at[pl.ds(r0, RPT)])
    plsc.subcore_barrier()
    base = (s * NC + c) * EPW

    def chunk(i, carry):
        off = base + i * CH
        pltpu.sync_copy(src_h.at[pl.ds(off, CH)], gidx)
        pltpu.sync_copy(dst_h.at[pl.ds(off, CH)], sidx)
        pltpu.async_copy(hwp_h.at[gidx], rows, sem).wait()
        pltpu.sync_copy(rows, acc.at[sidx], add=True)
        return carry

    lax.fori_loop(0, EPW // CH, chunk, 0)
    plsc.subcore_barrier()
    pltpu.sync_copy(acc.at[pl.ds(r0, RPT)], agg_h.at[c, pl.ds(r0, RPT)])


def _gcnout_body(a0_ref, a1_ref, hwp_ref, dinv_ref, bg_ref,
                 h_ref, s1_ref, s2_ref):
    i = pl.program_id(0)
    hb = (a0_ref[...] + a1_ref[...] + hwp_ref[...]) * dinv_ref[...] + bg_ref[...]
    hb = jnp.maximum(hb, 0.0)
    h_ref[...] = hb

    @pl.when(i == 0)
    def _():
        s1_ref[...] = jnp.zeros_like(s1_ref)
        s2_ref[...] = jnp.zeros_like(s2_ref)

    s1_ref[...] += jnp.sum(hb, axis=0, keepdims=True)
    s2_ref[...] += jnp.sum(hb * hb, axis=0, keepdims=True)


_gcnout_call = pl.pallas_call(
    _gcnout_body,
    grid=(NBLK,),
    in_specs=[_row_spec, _row_spec, _row_spec, _col_spec, _b_spec],
    out_specs=[_row_spec,
               pl.BlockSpec((1, F), lambda i: (0, 0)),
               pl.BlockSpec((1, F), lambda i: (0, 0))],
    out_shape=[jax.ShapeDtypeStruct((N, F), _f32),
               jax.ShapeDtypeStruct((1, F), _f32),
               jax.ShapeDtypeStruct((1, F), _f32)],
)


def _final_body(h_ref, s1_ref, s2_ref, gam_ref, bet_ref, wl_ref, bl_ref,
                y_ref):
    mu = s1_ref[...] / N
    var = s2_ref[...] / N - mu * mu
    inv = lax.rsqrt(var + 1e-5)
    gsc = gam_ref[...] * inv
    hb = h_ref[...] * gsc
    shift = bet_ref[...] - mu * gsc
    y_ref[...] = (jnp.dot(hb, wl_ref[...], preferred_element_type=_f32)
                  + jnp.dot(shift, wl_ref[...], preferred_element_type=_f32)
                  + bl_ref[...])


_final_call = pl.pallas_call(
    _final_body,
    grid=(NBLK,),
    in_specs=[_row_spec, _b_spec, _b_spec, _b_spec, _b_spec,
              pl.BlockSpec((F, 1), lambda i: (0, 0)),
              pl.BlockSpec((1, 1), lambda i: (0, 0))],
    out_specs=_col_spec,
    out_shape=jax.ShapeDtypeStruct((N, 1), _f32),
)


def kernel(x, edge_index, edge_weight, Wz, bz, Wr, br, Wh, bh, Wg, bg,
           gamma, beta, Wl, bl):
    src = edge_index[0]
    dst = edge_index[1]
    zn = jnp.zeros((NPAD,), _f32)
    zr = jnp.zeros((NPAD, F), _f32)

    dego, degi, cnt = _deg_kernel(src, dst, edge_weight, zn)
    doi2, dii2, dgc2 = _inv_call(dego, degi, cnt)
    doi = doi2.reshape(NPAD)
    dii = dii2.reshape(NPAD)
    dinv2 = dgc2[:, :N].reshape(N, 1)

    po_p, pi_p = _prop_kernel(x, src, dst, edge_weight, doi, dii, zr)
    po = po_p[:N]
    pi = pi_p[:N]

    hwp = _gru_call(x, po, pi,
                    Wz[0, 0, :F], Wz[1, 0, :F], Wz[0, 1, :F], Wz[1, 1, :F],
                    Wh[0, 0, :F], Wh[1, 0, :F], Wh[0, 1, :F], Wh[1, 1, :F],
                    bz.reshape(1, F), bh.reshape(1, F), Wg, dinv2)

    aggp = _gcn_kernel(hwp, src, dst, zr)

    h, s1, s2 = _gcnout_call(aggp[0, :N], aggp[1, :N], hwp, dinv2,
                             bg.reshape(1, F))

    y = _final_call(h, s1, s2, gamma.reshape(1, F), beta.reshape(1, F),
                    Wl, bl.reshape(1, 1))
    return y


# speedup vs baseline: 25.8622x; 2.3318x over previous
import functools

import jax
import jax.numpy as jnp
from jax import lax
from jax.experimental import pallas as pl
from jax.experimental.pallas import tpu as pltpu
from jax.experimental.pallas import tpu_sc as plsc

N = 10000
E = 320000
F = 128
NPAD = 10240
NC = 2
NS = 16
NW = NC * NS
CH = 80
EPW = E // NW
EPT = E // NS
CPW = EPW // CH
CPT = EPT // CH
RPT = NPAD // NS
SBP = 50
NSBP = CPT // SBP
PPS = SBP // 2
SB5 = 25
NSB5 = CPW // SB5
DR = NPAD // F
NBLK = 10
BR = N // NBLK
_f32 = jnp.float32
_i32 = jnp.int32

_mesh = plsc.VectorSubcoreMesh(core_axis_name="c", subcore_axis_name="s")
_sc_params = pltpu.CompilerParams(needs_layout_passes=False)


@functools.partial(
    pl.kernel,
    mesh=_mesh,
    out_type=[jax.ShapeDtypeStruct((NC, DR, F), _f32)] * 3,
    compiler_params=_sc_params,
    scratch_types=[
        pltpu.VMEM((CPW, CH), _i32),
        pltpu.VMEM((CPW, CH), _i32),
        pltpu.VMEM((CPW, CH), _f32),
        pltpu.VMEM((DR, F), _f32),
        pltpu.VMEM((DR, F), _f32),
        pltpu.VMEM((DR, F), _f32),
        pltpu.VMEM((DR,), _i32),
        pltpu.VMEM_SHARED((DR, F), _f32),
        pltpu.VMEM_SHARED((DR, F), _f32),
        pltpu.VMEM_SHARED((DR, F), _f32),
    ],
)
def _deg_kernel(src_h, dst_h, ew_h, zn2_h, dego_h, degi_h, cnt_h,
                sb, db, wb, dlo, dli, dlc, iob, acc_o, acc_i, acc_c):
    c = lax.axis_index("c")
    s = lax.axis_index("s")
    w = s * NC + c
    pltpu.sync_copy(src_h.at[w], sb)
    pltpu.sync_copy(dst_h.at[w], db)
    pltpu.sync_copy(ew_h.at[w], wb)
    pltpu.sync_copy(zn2_h, dlo)
    pltpu.sync_copy(zn2_h, dli)
    pltpu.sync_copy(zn2_h, dlc)
    for g in range(DR // 16):
        iob[pl.ds(g * 16, 16)] = jnp.arange(16, dtype=_i32) + (g * 16)

    @pl.when(s == 0)
    def _():
        pltpu.sync_copy(zn2_h, acc_o)
        pltpu.sync_copy(zn2_h, acc_i)
        pltpu.sync_copy(zn2_h, acc_c)

    ones16 = jnp.ones((16,), _f32)

    def row(r, carry):
        for g in range(CH // 16):
            si = sb[r, pl.ds(g * 16, 16)]
            di = db[r, pl.ds(g * 16, 16)]
            wv = wb[r, pl.ds(g * 16, 16)]
            plsc.addupdate_scatter(
                dlo,
                [lax.shift_right_logical(si, 7), jnp.bitwise_and(si, 127)],
                wv)
            plsc.addupdate_scatter(
                dli,
                [lax.shift_right_logical(di, 7), jnp.bitwise_and(di, 127)],
                wv)
            plsc.addupdate_scatter(
                dlc,
                [lax.shift_right_logical(di, 7), jnp.bitwise_and(di, 127)],
                ones16)
        return carry

    lax.fori_loop(0, CPW, row, 0)
    plsc.subcore_barrier()
    pltpu.sync_copy(dlo, acc_o.at[iob], add=True)
    pltpu.sync_copy(dli, acc_i.at[iob], add=True)
    pltpu.sync_copy(dlc, acc_c.at[iob], add=True)
    plsc.subcore_barrier()

    @pl.when(s == 0)
    def _():
        pltpu.sync_copy(acc_o, dego_h.at[c])
        pltpu.sync_copy(acc_i, degi_h.at[c])
        pltpu.sync_copy(acc_c, cnt_h.at[c])


def _inv_body(dego_ref, degi_ref, cnt_ref, doi_ref, dii_ref, dgc_ref):
    po = dego_ref[0:1, :] + dego_ref[1:2, :]
    pi = degi_ref[0:1, :] + degi_ref[1:2, :]
    pc = cnt_ref[0:1, :] + cnt_ref[1:2, :]
    doi_ref[...] = jnp.where(po > 0, 1.0 / po, 0.0)
    dii_ref[...] = jnp.where(pi > 0, 1.0 / pi, 0.0)
    dgc_ref[...] = lax.rsqrt(pc + 1.0)


_inv_call = pl.pallas_call(
    _inv_body,
    out_shape=[jax.ShapeDtypeStruct((1, NPAD), _f32)] * 3,
)


def _xscale_body(x_ref, dinv_ref, out_ref):
    out_ref[...] = (x_ref[...] * dinv_ref[0])[None]


_xscale_call = pl.pallas_call(
    _xscale_body,
    grid=(2, NBLK),
    in_specs=[pl.BlockSpec((BR, F), lambda j, i: (i, 0)),
              pl.BlockSpec((1, BR, 1), lambda j, i: (j, i, 0))],
    out_specs=pl.BlockSpec((1, BR, F), lambda j, i: (j, i, 0)),
    out_shape=jax.ShapeDtypeStruct((2, N, F), _f32),
)


@functools.partial(
    pl.kernel,
    mesh=_mesh,
    out_type=jax.ShapeDtypeStruct((NC, NPAD, F), _f32),
    compiler_params=_sc_params,
    scratch_types=[
        pltpu.VMEM((SBP, CH), _i32),
        pltpu.VMEM((SBP, CH), _i32),
        pltpu.VMEM((SBP, CH), _f32),
        pltpu.VMEM((CH,), _f32),
        pltpu.VMEM((CH, F), _f32),
        pltpu.VMEM((CH, F), _f32),
        pltpu.VMEM_SHARED((NPAD, F), _f32),
        pltpu.SemaphoreType.DMA,
        pltpu.SemaphoreType.DMA,
        pltpu.SemaphoreType.DMA,
        pltpu.SemaphoreType.DMA,
    ],
)
def _prop_kernel(xo_h, xi_h, src_h, dst_h, ew_h, zr_h, pp_h,
                 gix, six, nb, wrow, rows_a, rows_b, acc, gs_a, gs_b, ss_a,
                 ss_b):
    c = lax.axis_index("c")
    s = lax.axis_index("s")
    r0 = s * RPT
    pltpu.sync_copy(zr_h.at[pl.ds(r0, RPT)], acc.at[pl.ds(r0, RPT)])
    plsc.subcore_barrier()

    rows = (rows_a, rows_b)
    gsems = (gs_a, gs_b)
    ssems = (ss_a, ss_b)

    def start_gather(i, buf, sem):
        @pl.when(c == 0)
        def _():
            pltpu.async_copy(xo_h.at[gix.at[i]], buf, sem)

        @pl.when(c == 1)
        def _():
            pltpu.async_copy(xi_h.at[gix.at[i]], buf, sem)

    def wait_gather(i, buf, sem):
        pltpu.make_async_copy(xo_h.at[gix.at[i]], buf, sem).wait()

    def scale(buf, r):
        for g in range(CH // 16):
            wrow[pl.ds(g * 16, 16)] = nb[r, pl.ds(g * 16, 16)]

        def erow(e, carry2):
            nv = plsc.load_gather(wrow, [jnp.full((16,), e, _i32)])
            for sg in range(F // 16):
                buf[e, pl.ds(sg * 16, 16)] = buf[e, pl.ds(sg * 16, 16)] * nv
            return carry2

        lax.fori_loop(0, CH, erow, 0)

    def pairstep(p, carry):
        lp = p % PPS
        blk = s * NSBP + p // PPS

        @pl.when((lp == 0) & (p > 0))
        def _():
            pltpu.make_async_copy(rows_b, acc.at[six.at[0]], ss_b).wait()

        @pl.when((lp == 0) & (c == 0))
        def _():
            pltpu.sync_copy(src_h.at[blk], gix)
            pltpu.sync_copy(dst_h.at[blk], six)

        @pl.when((lp == 0) & (c == 1))
        def _():
            pltpu.sync_copy(dst_h.at[blk], gix)
            pltpu.sync_copy(src_h.at[blk], six)

        @pl.when(lp == 0)
        def _():
            pltpu.sync_copy(ew_h.at[blk], nb)
            start_gather(0, rows_a, gs_a)

        for b in (0, 1):
            i = 2 * lp + b
            bx, by = rows[b], rows[1 - b]

            @pl.when(i >= 1)
            def _():
                pltpu.make_async_copy(by, acc.at[six.at[i]],
                                      ssems[1 - b]).wait()

            @pl.when(i + 1 < SBP)
            def _():
                start_gather(i + 1, by, gsems[1 - b])

            wait_gather(i, bx, gsems[b])
            scale(bx, i)
            pltpu.async_copy(bx, acc.at[six.at[i]], ssems[b], add=True)
        return carry

    lax.fori_loop(0, CPT // 2, pairstep, 0)
    pltpu.make_async_copy(rows_b, acc.at[six.at[0]], ss_b).wait()
    plsc.subcore_barrier()
    pltpu.sync_copy(acc.at[pl.ds(r0, RPT)], pp_h.at[c, pl.ds(r0, RPT)])


def _gru_body(x_ref, po_ref, pi_ref, wz00_ref, wz10_ref, wz01_ref, wz11_ref,
              wh00_ref, wh10_ref, wh01_ref, wh11_ref, bz_ref, bh_ref,
              wg_ref, dinv_ref, out_ref):
    xb = x_ref[...]
    pob = po_ref[...]
    pib = pi_ref[...]
    az = wz00_ref[...] + wz10_ref[...]
    ah = wh00_ref[...] + wh10_ref[...]
    zpre = (jnp.dot(xb, az, preferred_element_type=_f32)
            + jnp.dot(pob, wz01_ref[...], preferred_element_type=_f32)
            + jnp.dot(pib, wz11_ref[...], preferred_element_type=_f32)
            + bz_ref[...])
    hpre = (jnp.dot(xb, ah, preferred_element_type=_f32)
            + jnp.dot(pob, wh01_ref[...], preferred_element_type=_f32)
            + jnp.dot(pib, wh11_ref[...], preferred_element_type=_f32)
            + bh_ref[...])
    z = jax.nn.sigmoid(zpre)
    ht = jnp.tanh(hpre)
    h = (1.0 - z) * ht
    out_ref[...] = dinv_ref[...] * jnp.dot(h, wg_ref[...],
                                           preferred_element_type=_f32)


_w_spec = pl.BlockSpec((F, F), lambda i: (0, 0))
_b_spec = pl.BlockSpec((1, F), lambda i: (0, 0))
_row_spec = pl.BlockSpec((BR, F), lambda i: (i, 0))
_col_spec = pl.BlockSpec((BR, 1), lambda i: (i, 0))

_gru_call = pl.pallas_call(
    _gru_body,
    grid=(NBLK,),
    in_specs=[_row_spec, _row_spec, _row_spec,
              _w_spec, _w_spec, _w_spec, _w_spec,
              _w_spec, _w_spec, _w_spec, _w_spec,
              _b_spec, _b_spec, _w_spec, _col_spec],
    out_specs=_row_spec,
    out_shape=jax.ShapeDtypeStruct((N, F), _f32),
)


@functools.partial(
    pl.kernel,
    mesh=_mesh,
    out_type=jax.ShapeDtypeStruct((NC, NPAD, F), _f32),
    compiler_params=_sc_params,
    scratch_types=[
        pltpu.VMEM((SB5, CH), _i32),
        pltpu.VMEM((SB5, CH), _i32),
        pltpu.VMEM((CH, F), _f32),
        pltpu.VMEM((CH, F), _f32),
        pltpu.VMEM_SHARED((NPAD, F), _f32),
        pltpu.SemaphoreType.DMA,
        pltpu.SemaphoreType.DMA,
        pltpu.SemaphoreType.DMA,
        pltpu.SemaphoreType.DMA,
    ],
)
def _gcn_kernel(hwp_h, src_h, dst_h, zr_h, agg_h,
                gix, six, rows_a, rows_b, acc, gs_a, gs_b, ss_a, ss_b):
    c = lax.axis_index("c")
    s = lax.axis_index("s")
    r0 = s * RPT
    pltpu.sync_copy(zr_h.at[pl.ds(r0, RPT)], acc.at[pl.ds(r0, RPT)])
    plsc.subcore_barrier()

    rows = (rows_a, rows_b)
    gsems = (gs_a, gs_b)
    ssems = (ss_a, ss_b)

    def superblock(k, carry):
        blk = (s * NC + c) * NSB5 + k
        pltpu.sync_copy(src_h.at[blk], gix)
        pltpu.sync_copy(dst_h.at[blk], six)
        pltpu.async_copy(hwp_h.at[gix.at[0]], rows_a, gs_a)

        def pair(g, carry2):
            for b in (0, 1):
                i = 2 * g + b
                bx, by = rows[b], rows[1 - b]

                @pl.when(i >= 1)
                def _():
                    pltpu.make_async_copy(by, acc.at[six.at[i]],
                                          ssems[1 - b]).wait()

                pltpu.async_copy(hwp_h.at[gix.at[i + 1]], by, gsems[1 - b])
                pltpu.make_async_copy(hwp_h.at[gix.at[i]], bx,
                                      gsems[b]).wait()
                pltpu.async_copy(bx, acc.at[six.at[i]], ssems[b], add=True)
            return carry2

        lax.fori_loop(0, SB5 // 2, pair, 0)
        t = SB5 - 1
        pltpu.make_async_copy(rows_b, acc.at[six.at[t]], ss_b).wait()
        pltpu.make_async_copy(hwp_h.at[gix.at[t]], rows_a, gs_a).wait()
        pltpu.async_copy(rows_a, acc.at[six.at[t]], ss_a, add=True)
        pltpu.make_async_copy(rows_a, acc.at[six.at[t]], ss_a).wait()
        return carry

    lax.fori_loop(0, NSB5, superblock, 0)
    plsc.subcore_barrier()
    pltpu.sync_copy(acc.at[pl.ds(r0, RPT)], agg_h.at[c, pl.ds(r0, RPT)])


def _gcnout_body(a0_ref, a1_ref, hwp_ref, dinv_ref, bg_ref,
                 h_ref, s1_ref, s2_ref):
    i = pl.program_id(0)
    hb = (a0_ref[...] + a1_ref[...] + hwp_ref[...]) * dinv_ref[...] + bg_ref[...]
    hb = jnp.maximum(hb, 0.0)
    h_ref[...] = hb

    @pl.when(i == 0)
    def _():
        s1_ref[...] = jnp.zeros_like(s1_ref)
        s2_ref[...] = jnp.zeros_like(s2_ref)

    s1_ref[...] += jnp.sum(hb, axis=0, keepdims=True)
    s2_ref[...] += jnp.sum(hb * hb, axis=0, keepdims=True)


_gcnout_call = pl.pallas_call(
    _gcnout_body,
    grid=(NBLK,),
    in_specs=[_row_spec, _row_spec, _row_spec, _col_spec, _b_spec],
    out_specs=[_row_spec,
               pl.BlockSpec((1, F), lambda i: (0, 0)),
               pl.BlockSpec((1, F), lambda i: (0, 0))],
    out_shape=[jax.ShapeDtypeStruct((N, F), _f32),
               jax.ShapeDtypeStruct((1, F), _f32),
               jax.ShapeDtypeStruct((1, F), _f32)],
)


def _final_body(h_ref, s1_ref, s2_ref, gam_ref, bet_ref, wl_ref, bl_ref,
                y_ref):
    mu = s1_ref[...] / N
    var = s2_ref[...] / N - mu * mu
    inv = lax.rsqrt(var + 1e-5)
    gsc = gam_ref[...] * inv
    hb = h_ref[...] * gsc
    shift = bet_ref[...] - mu * gsc
    y_ref[...] = (jnp.dot(hb, wl_ref[...], preferred_element_type=_f32)
                  + jnp.dot(shift, wl_ref[...], preferred_element_type=_f32)
                  + bl_ref[...])


_final_call = pl.pallas_call(
    _final_body,
    grid=(NBLK,),
    in_specs=[_row_spec, _b_spec, _b_spec, _b_spec, _b_spec,
              pl.BlockSpec((F, 1), lambda i: (0, 0)),
              pl.BlockSpec((1, 1), lambda i: (0, 0))],
    out_specs=_col_spec,
    out_shape=jax.ShapeDtypeStruct((N, 1), _f32),
)


def kernel(x, edge_index, edge_weight, Wz, bz, Wr, br, Wh, bh, Wg, bg,
           gamma, beta, Wl, bl):
    src = edge_index[0]
    dst = edge_index[1]
    src_w = src.reshape(NW, CPW, CH)
    dst_w = dst.reshape(NW, CPW, CH)
    ew_w = edge_weight.reshape(NW, CPW, CH)
    src_t = src.reshape(NS * NSBP, SBP, CH)
    dst_t = dst.reshape(NS * NSBP, SBP, CH)
    ew_t = edge_weight.reshape(NS * NSBP, SBP, CH)
    src_5 = src.reshape(NW * NSB5, SB5, CH)
    dst_5 = dst.reshape(NW * NSB5, SB5, CH)
    zn2 = jnp.zeros((DR, F), _f32)
    zr = jnp.zeros((NPAD, F), _f32)

    dego, degi, cnt = _deg_kernel(src_w, dst_w, ew_w, zn2)
    doi2, dii2, dgc2 = _inv_call(dego.reshape(NC, NPAD),
                                 degi.reshape(NC, NPAD),
                                 cnt.reshape(NC, NPAD))
    dinv2 = dgc2[:, :N].reshape(N, 1)
    dcat = jnp.stack([doi2[0, :N].reshape(N, 1), dii2[0, :N].reshape(N, 1)])

    xcat = _xscale_call(x, dcat)
    pp = _prop_kernel(xcat[0], xcat[1], src_t, dst_t, ew_t, zr)
    po = pp[0, :N]
    pi = pp[1, :N]

    hwp = _gru_call(x, po, pi,
                    Wz[0, 0, :F], Wz[1, 0, :F], Wz[0, 1, :F], Wz[1, 1, :F],
                    Wh[0, 0, :F], Wh[1, 0, :F], Wh[0, 1, :F], Wh[1, 1, :F],
                    bz.reshape(1, F), bh.reshape(1, F), Wg, dinv2)

    aggp = _gcn_kernel(hwp, src_5, dst_5, zr)

    h, s1, s2 = _gcnout_call(aggp[0, :N], aggp[1, :N], hwp, dinv2,
                             bg.reshape(1, F))

    y = _final_call(h, s1, s2, gamma.reshape(1, F), beta.reshape(1, F),
                    Wl, bl.reshape(1, 1))
    return y


# speedup vs baseline: 27.4995x; 1.0633x over previous
import functools

import jax
import jax.numpy as jnp
from jax import lax
from jax.experimental import pallas as pl
from jax.experimental.pallas import tpu as pltpu
from jax.experimental.pallas import tpu_sc as plsc

N = 10000
E = 320000
F = 128
NPAD = 10240
NC = 2
NS = 16
NW = NC * NS
CH = 80
EPW = E // NW
EPT = E // NS
CPW = EPW // CH
CPT = EPT // CH
RPT = NPAD // NS
SBP = 50
NSBP = CPT // SBP
PPS = SBP // 2
SB5 = 25
NSB5 = CPW // SB5
DR = NPAD // F
NBLK = 10
BR = N // NBLK
_f32 = jnp.float32
_i32 = jnp.int32

_mesh = plsc.VectorSubcoreMesh(core_axis_name="c", subcore_axis_name="s")
_sc_params = pltpu.CompilerParams(needs_layout_passes=False)


@functools.partial(
    pl.kernel,
    mesh=_mesh,
    out_type=[jax.ShapeDtypeStruct((NC, DR, F), _f32)] * 3,
    compiler_params=_sc_params,
    scratch_types=[
        pltpu.VMEM((CPW, CH), _i32),
        pltpu.VMEM((CPW, CH), _i32),
        pltpu.VMEM((CPW, CH), _f32),
        pltpu.VMEM((DR, F), _f32),
        pltpu.VMEM((DR, F), _f32),
        pltpu.VMEM((DR, F), _f32),
        pltpu.VMEM((DR,), _i32),
        pltpu.VMEM_SHARED((DR, F), _f32),
        pltpu.VMEM_SHARED((DR, F), _f32),
        pltpu.VMEM_SHARED((DR, F), _f32),
    ],
)
def _deg_kernel(src_h, dst_h, ew_h, zn2_h, dego_h, degi_h, cnt_h,
                sb, db, wb, dlo, dli, dlc, iob, acc_o, acc_i, acc_c):
    c = lax.axis_index("c")
    s = lax.axis_index("s")
    w = s * NC + c
    pltpu.sync_copy(src_h.at[w], sb)
    pltpu.sync_copy(dst_h.at[w], db)
    pltpu.sync_copy(ew_h.at[w], wb)
    pltpu.sync_copy(zn2_h, dlo)
    pltpu.sync_copy(zn2_h, dli)
    pltpu.sync_copy(zn2_h, dlc)
    for g in range(DR // 16):
        iob[pl.ds(g * 16, 16)] = jnp.arange(16, dtype=_i32) + (g * 16)

    @pl.when(s == 0)
    def _():
        pltpu.sync_copy(zn2_h, acc_o)
        pltpu.sync_copy(zn2_h, acc_i)
        pltpu.sync_copy(zn2_h, acc_c)

    ones16 = jnp.ones((16,), _f32)

    def row(r, carry):
        for g in range(CH // 16):
            si = sb[r, pl.ds(g * 16, 16)]
            di = db[r, pl.ds(g * 16, 16)]
            wv = wb[r, pl.ds(g * 16, 16)]
            plsc.addupdate_scatter(
                dlo,
                [lax.shift_right_logical(si, 7), jnp.bitwise_and(si, 127)],
                wv)
            plsc.addupdate_scatter(
                dli,
                [lax.shift_right_logical(di, 7), jnp.bitwise_and(di, 127)],
                wv)
            plsc.addupdate_scatter(
                dlc,
                [lax.shift_right_logical(di, 7), jnp.bitwise_and(di, 127)],
                ones16)
        return carry

    lax.fori_loop(0, CPW, row, 0)
    plsc.subcore_barrier()
    pltpu.sync_copy(dlo, acc_o.at[iob], add=True)
    pltpu.sync_copy(dli, acc_i.at[iob], add=True)
    pltpu.sync_copy(dlc, acc_c.at[iob], add=True)
    plsc.subcore_barrier()

    @pl.when(s == 0)
    def _():
        pltpu.sync_copy(acc_o, dego_h.at[c])
        pltpu.sync_copy(acc_i, degi_h.at[c])
        pltpu.sync_copy(acc_c, cnt_h.at[c])


def _pre_body(x_ref, degp_ref, xo_ref, xi_ref, dgc_ref):
    do = degp_ref[0, 0] + degp_ref[0, 1]
    di = degp_ref[1, 0] + degp_ref[1, 1]
    dc = degp_ref[2, 0] + degp_ref[2, 1]
    xb = x_ref[...]
    xo_ref[...] = xb * jnp.where(do > 0, 1.0 / do, 0.0)
    xi_ref[...] = xb * jnp.where(di > 0, 1.0 / di, 0.0)
    dgc_ref[...] = lax.rsqrt(dc + 1.0)


_pre_call = pl.pallas_call(
    _pre_body,
    grid=(NBLK,),
    in_specs=[pl.BlockSpec((BR, F), lambda i: (i, 0)),
              pl.BlockSpec((3, NC, BR, 1), lambda i: (0, 0, i, 0))],
    out_specs=[pl.BlockSpec((BR, F), lambda i: (i, 0)),
               pl.BlockSpec((BR, F), lambda i: (i, 0)),
               pl.BlockSpec((BR, 1), lambda i: (i, 0))],
    out_shape=[jax.ShapeDtypeStruct((N, F), _f32),
               jax.ShapeDtypeStruct((N, F), _f32),
               jax.ShapeDtypeStruct((N, 1), _f32)],
)


@functools.partial(
    pl.kernel,
    mesh=_mesh,
    out_type=jax.ShapeDtypeStruct((NC, NPAD, F), _f32),
    compiler_params=_sc_params,
    scratch_types=[
        pltpu.VMEM((SBP, CH), _i32),
        pltpu.VMEM((SBP, CH), _i32),
        pltpu.VMEM((SBP, CH), _f32),
        pltpu.VMEM((CH,), _f32),
        pltpu.VMEM((CH, F), _f32),
        pltpu.VMEM((CH, F), _f32),
        pltpu.VMEM_SHARED((NPAD, F), _f32),
        pltpu.SemaphoreType.DMA,
        pltpu.SemaphoreType.DMA,
        pltpu.SemaphoreType.DMA,
        pltpu.SemaphoreType.DMA,
    ],
)
def _prop_kernel(xo_h, xi_h, src_h, dst_h, ew_h, zr_h, pp_h,
                 gix, six, nb, wrow, rows_a, rows_b, acc, gs_a, gs_b, ss_a,
                 ss_b):
    c = lax.axis_index("c")
    s = lax.axis_index("s")
    r0 = s * RPT
    pltpu.sync_copy(zr_h.at[pl.ds(r0, RPT)], acc.at[pl.ds(r0, RPT)])
    plsc.subcore_barrier()

    rows = (rows_a, rows_b)
    gsems = (gs_a, gs_b)
    ssems = (ss_a, ss_b)

    def start_gather(i, buf, sem):
        @pl.when(c == 0)
        def _():
            pltpu.async_copy(xo_h.at[gix.at[i]], buf, sem)

        @pl.when(c == 1)
        def _():
            pltpu.async_copy(xi_h.at[gix.at[i]], buf, sem)

    def wait_gather(i, buf, sem):
        pltpu.make_async_copy(xo_h.at[gix.at[i]], buf, sem).wait()

    def scale(buf, r):
        for g in range(CH // 16):
            wrow[pl.ds(g * 16, 16)] = nb[r, pl.ds(g * 16, 16)]

        def egrp(g2, carry2):
            base = g2 * 16
            for j in range(16):
                e = base + j
                nv = plsc.load_gather(wrow, [jnp.full((16,), e, _i32)])
                for sg in range(F // 16):
                    buf[e, pl.ds(sg * 16, 16)] = (
                        buf[e, pl.ds(sg * 16, 16)] * nv)
            return carry2

        lax.fori_loop(0, CH // 16, egrp, 0)

    def pairstep(p, carry):
        lp = p % PPS
        blk = s * NSBP + p // PPS

        @pl.when((lp == 0) & (p > 0))
        def _():
            pltpu.make_async_copy(rows_b, acc.at[six.at[0]], ss_b).wait()

        @pl.when((lp == 0) & (c == 0))
        def _():
            pltpu.sync_copy(src_h.at[blk], gix)
            pltpu.sync_copy(dst_h.at[blk], six)

        @pl.when((lp == 0) & (c == 1))
        def _():
            pltpu.sync_copy(dst_h.at[blk], gix)
            pltpu.sync_copy(src_h.at[blk], six)

        @pl.when(lp == 0)
        def _():
            pltpu.sync_copy(ew_h.at[blk], nb)
            start_gather(0, rows_a, gs_a)

        for b in (0, 1):
            i = 2 * lp + b
            bx, by = rows[b], rows[1 - b]

            @pl.when(i >= 1)
            def _():
                pltpu.make_async_copy(by, acc.at[six.at[i]],
                                      ssems[1 - b]).wait()

            @pl.when(i + 1 < SBP)
            def _():
                start_gather(i + 1, by, gsems[1 - b])

            wait_gather(i, bx, gsems[b])
            scale(bx, i)
            pltpu.async_copy(bx, acc.at[six.at[i]], ssems[b], add=True)
        return carry

    lax.fori_loop(0, CPT // 2, pairstep, 0)
    pltpu.make_async_copy(rows_b, acc.at[six.at[0]], ss_b).wait()
    plsc.subcore_barrier()
    pltpu.sync_copy(acc.at[pl.ds(r0, RPT)], pp_h.at[c, pl.ds(r0, RPT)])


def _gru_body(x_ref, po_ref, pi_ref, wz00_ref, wz10_ref, wz01_ref, wz11_ref,
              wh00_ref, wh10_ref, wh01_ref, wh11_ref, bz_ref, bh_ref,
              wg_ref, dinv_ref, out_ref):
    xb = x_ref[...]
    pob = po_ref[0]
    pib = pi_ref[0]
    az = wz00_ref[...] + wz10_ref[...]
    ah = wh00_ref[...] + wh10_ref[...]
    zpre = (jnp.dot(xb, az, preferred_element_type=_f32)
            + jnp.dot(pob, wz01_ref[...], preferred_element_type=_f32)
            + jnp.dot(pib, wz11_ref[...], preferred_element_type=_f32)
            + bz_ref[...])
    hpre = (jnp.dot(xb, ah, preferred_element_type=_f32)
            + jnp.dot(pob, wh01_ref[...], preferred_element_type=_f32)
            + jnp.dot(pib, wh11_ref[...], preferred_element_type=_f32)
            + bh_ref[...])
    z = jax.nn.sigmoid(zpre)
    ht = jnp.tanh(hpre)
    h = (1.0 - z) * ht
    out_ref[...] = dinv_ref[...] * jnp.dot(h, wg_ref[...],
                                           preferred_element_type=_f32)


_w_spec = pl.BlockSpec((F, F), lambda i: (0, 0))
_b_spec = pl.BlockSpec((1, F), lambda i: (0, 0))
_row_spec = pl.BlockSpec((BR, F), lambda i: (i, 0))
_col_spec = pl.BlockSpec((BR, 1), lambda i: (i, 0))

_po_spec = pl.BlockSpec((1, BR, F), lambda i: (0, i, 0))
_pi_spec = pl.BlockSpec((1, BR, F), lambda i: (1, i, 0))

_gru_call = pl.pallas_call(
    _gru_body,
    grid=(NBLK,),
    in_specs=[_row_spec, _po_spec, _pi_spec,
              _w_spec, _w_spec, _w_spec, _w_spec,
              _w_spec, _w_spec, _w_spec, _w_spec,
              _b_spec, _b_spec, _w_spec, _col_spec],
    out_specs=_row_spec,
    out_shape=jax.ShapeDtypeStruct((N, F), _f32),
)


@functools.partial(
    pl.kernel,
    mesh=_mesh,
    out_type=jax.ShapeDtypeStruct((NC, NPAD, F), _f32),
    compiler_params=_sc_params,
    scratch_types=[
        pltpu.VMEM((SB5, CH), _i32),
        pltpu.VMEM((SB5, CH), _i32),
        pltpu.VMEM((CH, F), _f32),
        pltpu.VMEM((CH, F), _f32),
        pltpu.VMEM_SHARED((NPAD, F), _f32),
        pltpu.SemaphoreType.DMA,
        pltpu.SemaphoreType.DMA,
        pltpu.SemaphoreType.DMA,
        pltpu.SemaphoreType.DMA,
    ],
)
def _gcn_kernel(hwp_h, src_h, dst_h, zr_h, agg_h,
                gix, six, rows_a, rows_b, acc, gs_a, gs_b, ss_a, ss_b):
    c = lax.axis_index("c")
    s = lax.axis_index("s")
    r0 = s * RPT
    pltpu.sync_copy(zr_h.at[pl.ds(r0, RPT)], acc.at[pl.ds(r0, RPT)])
    plsc.subcore_barrier()

    rows = (rows_a, rows_b)
    gsems = (gs_a, gs_b)
    ssems = (ss_a, ss_b)

    def superblock(k, carry):
        blk = (s * NC + c) * NSB5 + k
        pltpu.sync_copy(src_h.at[blk], gix)
        pltpu.sync_copy(dst_h.at[blk], six)
        pltpu.async_copy(hwp_h.at[gix.at[0]], rows_a, gs_a)

        def pair(g, carry2):
            for b in (0, 1):
                i = 2 * g + b
                bx, by = rows[b], rows[1 - b]

                @pl.when(i >= 1)
                def _():
                    pltpu.make_async_copy(by, acc.at[six.at[i]],
                                          ssems[1 - b]).wait()

                pltpu.async_copy(hwp_h.at[gix.at[i + 1]], by, gsems[1 - b])
                pltpu.make_async_copy(hwp_h.at[gix.at[i]], bx,
                                      gsems[b]).wait()
                pltpu.async_copy(bx, acc.at[six.at[i]], ssems[b], add=True)
            return carry2

        lax.fori_loop(0, SB5 // 2, pair, 0)
        t = SB5 - 1
        pltpu.make_async_copy(rows_b, acc.at[six.at[t]], ss_b).wait()
        pltpu.make_async_copy(hwp_h.at[gix.at[t]], rows_a, gs_a).wait()
        pltpu.async_copy(rows_a, acc.at[six.at[t]], ss_a, add=True)
        pltpu.make_async_copy(rows_a, acc.at[six.at[t]], ss_a).wait()
        return carry

    lax.fori_loop(0, NSB5, superblock, 0)
    plsc.subcore_barrier()
    pltpu.sync_copy(acc.at[pl.ds(r0, RPT)], agg_h.at[c, pl.ds(r0, RPT)])


def _gcnout_body(a0_ref, a1_ref, hwp_ref, dinv_ref, bg_ref,
                 h_ref, s1_ref, s2_ref):
    i = pl.program_id(0)
    hb = (a0_ref[0] + a1_ref[0] + hwp_ref[...]) * dinv_ref[...] + bg_ref[...]
    hb = jnp.maximum(hb, 0.0)
    h_ref[...] = hb

    @pl.when(i == 0)
    def _():
        s1_ref[...] = jnp.zeros_like(s1_ref)
        s2_ref[...] = jnp.zeros_like(s2_ref)

    s1_ref[...] += jnp.sum(hb, axis=0, keepdims=True)
    s2_ref[...] += jnp.sum(hb * hb, axis=0, keepdims=True)


_gcnout_call = pl.pallas_call(
    _gcnout_body,
    grid=(NBLK,),
    in_specs=[_po_spec, _pi_spec, _row_spec, _col_spec, _b_spec],
    out_specs=[_row_spec,
               pl.BlockSpec((1, F), lambda i: (0, 0)),
               pl.BlockSpec((1, F), lambda i: (0, 0))],
    out_shape=[jax.ShapeDtypeStruct((N, F), _f32),
               jax.ShapeDtypeStruct((1, F), _f32),
               jax.ShapeDtypeStruct((1, F), _f32)],
)


def _final_body(h_ref, s1_ref, s2_ref, gam_ref, bet_ref, wl_ref, bl_ref,
                y_ref):
    mu = s1_ref[...] / N
    var = s2_ref[...] / N - mu * mu
    inv = lax.rsqrt(var + 1e-5)
    gsc = gam_ref[...] * inv
    hb = h_ref[...] * gsc
    shift = bet_ref[...] - mu * gsc
    y_ref[...] = (jnp.dot(hb, wl_ref[...], preferred_element_type=_f32)
                  + jnp.dot(shift, wl_ref[...], preferred_element_type=_f32)
                  + bl_ref[...])


_final_call = pl.pallas_call(
    _final_body,
    grid=(NBLK,),
    in_specs=[_row_spec, _b_spec, _b_spec, _b_spec, _b_spec,
              pl.BlockSpec((F, 1), lambda i: (0, 0)),
              pl.BlockSpec((1, 1), lambda i: (0, 0))],
    out_specs=_col_spec,
    out_shape=jax.ShapeDtypeStruct((N, 1), _f32),
)


def kernel(x, edge_index, edge_weight, Wz, bz, Wr, br, Wh, bh, Wg, bg,
           gamma, beta, Wl, bl):
    src = edge_index[0]
    dst = edge_index[1]
    src_w = src.reshape(NW, CPW, CH)
    dst_w = dst.reshape(NW, CPW, CH)
    ew_w = edge_weight.reshape(NW, CPW, CH)
    src_t = src.reshape(NS * NSBP, SBP, CH)
    dst_t = dst.reshape(NS * NSBP, SBP, CH)
    ew_t = edge_weight.reshape(NS * NSBP, SBP, CH)
    src_5 = src.reshape(NW * NSB5, SB5, CH)
    dst_5 = dst.reshape(NW * NSB5, SB5, CH)
    zn2 = jnp.zeros((DR, F), _f32)
    zr = jnp.zeros((NPAD, F), _f32)

    dego, degi, cnt = _deg_kernel(src_w, dst_w, ew_w, zn2)
    degp = jnp.stack([dego.reshape(NC, NPAD), degi.reshape(NC, NPAD),
                      cnt.reshape(NC, NPAD)]).reshape(3, NC, NPAD, 1)
    xo, xi, dinv2 = _pre_call(x, degp)

    pp = _prop_kernel(xo, xi, src_t, dst_t, ew_t, zr)

    hwp = _gru_call(x, pp, pp,
                    Wz[0, 0, :F], Wz[1, 0, :F], Wz[0, 1, :F], Wz[1, 1, :F],
                    Wh[0, 0, :F], Wh[1, 0, :F], Wh[0, 1, :F], Wh[1, 1, :F],
                    bz.reshape(1, F), bh.reshape(1, F), Wg, dinv2)

    aggp = _gcn_kernel(hwp, src_5, dst_5, zr)

    h, s1, s2 = _gcnout_call(aggp, aggp, hwp, dinv2, bg.reshape(1, F))

    y = _final_call(h, s1, s2, gamma.reshape(1, F), beta.reshape(1, F),
                    Wl, bl.reshape(1, 1))
    return y
